# Initial kernel scaffold; baseline (speedup 1.0000x reference)
import jax
import jax.numpy as jnp
from jax.experimental import pallas as pl


def kernel(input_ids, numeric_values, numeric_formats, word_emb, pos_emb, type_emb, ln_g, ln_b, w1, b1, w2, b2, nln_g, nln_b, gate_w, gate_b):
    raise NotImplementedError("write your pallas kernel here")


# speedup vs baseline: 2.2754x; 2.2754x over previous
import functools
import math

import jax
import jax.numpy as jnp
from jax import lax
from jax.experimental import pallas as pl
from jax.experimental.pallas import tpu as pltpu
from jax.experimental.pallas import tpu_sc as plsc

_NUM_TOKEN_ID = 5
_EPS = 1e-12
_NBITS = 16
_BLK = 512
_SC_CHUNK = 128


def _sc_gather(word_emb, ids_flat):
    n = ids_flat.shape[0]
    h = word_emb.shape[1]
    info = plsc.get_sparse_core_info()
    nc, ns = info.num_cores, info.num_subcores
    nw = nc * ns
    bpw = n // nw
    n_chunks = bpw // _SC_CHUNK
    mesh = plsc.VectorSubcoreMesh(core_axis_name="c", subcore_axis_name="s")

    @functools.partial(
        pl.kernel,
        mesh=mesh,
        out_type=jax.ShapeDtypeStruct((n, h), jnp.float32),
        scratch_types=[
            pltpu.VMEM((bpw,), jnp.int32),
            pltpu.VMEM((_SC_CHUNK, h), jnp.float32),
            pltpu.SemaphoreType.DMA,
        ],
    )
    def k(table_hbm, idx_hbm, out_hbm, idx_v, rows_v, sem):
        wid = lax.axis_index("s") * nc + lax.axis_index("c")
        base = wid * bpw
        pltpu.sync_copy(idx_hbm.at[pl.ds(base, bpw)], idx_v)

        def body(j, carry):
            off = j * _SC_CHUNK
            pltpu.async_copy(
                table_hbm.at[idx_v.at[pl.ds(off, _SC_CHUNK)]], rows_v, sem
            ).wait()
            pltpu.sync_copy(rows_v, out_hbm.at[pl.ds(base + off, _SC_CHUNK)])
            return carry

        lax.fori_loop(0, n_chunks, body, 0)

    return k(word_emb, ids_flat)


def _ln(x, g, b):
    m = jnp.mean(x, axis=-1, keepdims=True)
    d = x - m
    v = jnp.mean(d * d, axis=-1, keepdims=True)
    return d * lax.rsqrt(v + _EPS) * g + b


def _fused_body(tw_ref, posw_ref, ids_ref, vals_ref, fmt_ref, bin_ref,
                w1r_ref, w1bin_ref, b1_ref, w2_ref, b2_ref,
                nlng_ref, nlnb_ref, gt_ref, gn_ref, gb_ref,
                lng_ref, lnb_ref, o_ref, acc_ref):
    ids = ids_ref[...]
    text = tw_ref[...] + posw_ref[...]
    acc_ref[...] = text
    has_num = jnp.any(ids == _NUM_TOKEN_ID)

    @pl.when(has_num)
    def _():
        v = vals_ref[...]
        av = jnp.abs(v)
        log_abs = jnp.log(av + 1e-6)
        sign = jnp.sign(v)
        expo = jnp.where(av > 1e-6,
                         jnp.floor(jnp.log(av + 1e-30) * (1.0 / math.log(10.0))),
                         0.0)
        fmt = fmt_ref[...]
        w1r = w1r_ref[...]
        h = (log_abs * w1r[0:1, :]
             + sign * w1r[1:2, :]
             + expo * w1r[2:3, :]
             + jnp.dot(bin_ref[...], w1bin_ref[...],
                       preferred_element_type=jnp.float32)
             + jnp.where(fmt == 0, 1.0, 0.0) * w1r[3:4, :]
             + jnp.where(fmt == 1, 1.0, 0.0) * w1r[4:5, :]
             + jnp.where(fmt == 2, 1.0, 0.0) * w1r[5:6, :]
             + b1_ref[...])
        h = 0.5 * h * (1.0 + lax.erf(h * (1.0 / math.sqrt(2.0))))
        h2 = jnp.dot(h, w2_ref[...], preferred_element_type=jnp.float32) \
            + b2_ref[...]
        num = _ln(h2, nlng_ref[...], nlnb_ref[...])
        text_l = acc_ref[...]
        gate = jax.nn.sigmoid(
            jnp.dot(text_l, gt_ref[...], preferred_element_type=jnp.float32)
            + jnp.dot(num, gn_ref[...], preferred_element_type=jnp.float32)
            + gb_ref[...])
        fused = gate * num + (1.0 - gate) * text_l
        acc_ref[...] = jnp.where(ids == _NUM_TOKEN_ID, fused, text_l)

    o_ref[...] = _ln(acc_ref[...], lng_ref[...], lnb_ref[...])


def _fused_tc(tw, posw, ids_col, vals_col, fmt_col, binf,
              w1rows, w1bin, b1, w2, b2, nln_g, nln_b, gt, gn, gb, ln_g, ln_b):
    n, h = tw.shape
    inter = w2.shape[0]
    grid = n // _BLK
    full = lambda r, c: pl.BlockSpec((r, c), lambda i: (0, 0))
    blk = lambda c, dt=None: pl.BlockSpec((_BLK, c), lambda i: (i, 0))
    return pl.pallas_call(
        _fused_body,
        grid=(grid,),
        in_specs=[
            blk(h),
            pl.BlockSpec((_BLK, h), lambda i: (i % (posw.shape[0] // _BLK), 0)),
            blk(1),
            blk(1),
            blk(1),
            blk(_NBITS),
            full(8, inter),
            full(_NBITS, inter),
            full(1, inter),
            full(inter, h),
            full(1, h),
            full(1, h), full(1, h),
            full(h, h), full(h, h),
            full(1, h),
            full(1, h), full(1, h),
        ],
        out_specs=blk(h),
        out_shape=jax.ShapeDtypeStruct((n, h), jnp.float32),
        scratch_shapes=[pltpu.VMEM((_BLK, h), jnp.float32)],
    )(tw, posw, ids_col, vals_col, fmt_col, binf,
      w1rows, w1bin, b1, w2, b2, nln_g, nln_b, gt, gn, gb, ln_g, ln_b)


def kernel(input_ids, numeric_values, numeric_formats, word_emb, pos_emb,
           type_emb, ln_g, ln_b, w1, b1, w2, b2, nln_g, nln_b, gate_w, gate_b):
    b, s = input_ids.shape
    n = b * s
    hid = word_emb.shape[1]
    inter = w1.shape[1]

    ids_flat = input_ids.reshape(n).astype(jnp.int32)
    tw = _sc_gather(word_emb, ids_flat)

    posw = pos_emb[:s] + type_emb[0][None, :]
    ids_col = ids_flat.reshape(n, 1)
    vals_col = numeric_values.reshape(n, 1)
    fmt_col = numeric_formats.reshape(n, 1).astype(jnp.int32)
    binf = jax.random.normal(jax.random.key(1), (b, s, _NBITS),
                             dtype=jnp.float32).reshape(n, _NBITS)
    w1rows = jnp.concatenate(
        [w1[0:3], w1[3 + _NBITS:], jnp.zeros((2, inter), jnp.float32)], axis=0)
    w1bin = w1[3:3 + _NBITS]
    gt = gate_w[:hid]
    gn = gate_w[hid:]
    row = lambda x: x.reshape(1, -1)

    out = _fused_tc(tw, posw, ids_col, vals_col, fmt_col, binf,
                    w1rows, w1bin, row(b1), w2, row(b2),
                    row(nln_g), row(nln_b), gt, gn, row(gate_b),
                    row(ln_g), row(ln_b))
    return out.reshape(b, s, hid)
